# baseline (speedup 1.0000x reference)
import jax
import jax.numpy as jnp
from jax import lax
from jax.experimental import pallas as pl
from jax.experimental.pallas import tpu as pltpu
from jax.experimental.pallas import tpu_sc as plsc

N_NODES = 50000
DIM = 64
N_EDGES = 800000

NC = 2
NS = 16
NW = NC * NS
HALF = N_NODES // NC
DUMMY = HALF
RPT = 1568
ACC_ROWS = RPT * NS
RPT_LAST = HALF - (NS - 1) * RPT
K = 128
CH_W = 3 * K
NBUF = 3
PEPT = N_EDGES // NW
SEGCH = (PEPT + K - 1) // K
BIN = 1000
NBLK = PEPT // BIN
STSZ = 272
TRASH = 256

_mesh = plsc.VectorSubcoreMesh(core_axis_name="c", subcore_axis_name="s")


def _partition_body(row, col, w, edata, ncnk,
                    inr, inc, inw, st0r, st0c, st0w, st1r, st1c, st1w, cbuf):
    c = lax.axis_index("c")
    s = lax.axis_index("s")
    t = c * NS + s
    ebase = t * PEPT

    iota = lax.iota(jnp.int32, 16)

    def seg_off(bkt):
        return (bkt * NW + t) * SEGCH * CH_W

    def flush_chunk(bkt, sr, sc, sw, nch):
        off = seg_off(bkt) + nch * CH_W
        pltpu.sync_copy(sr.at[pl.ds(0, K)], edata.at[pl.ds(off, K)])
        pltpu.sync_copy(sc.at[pl.ds(0, K)], edata.at[pl.ds(off + K, K)])
        pltpu.sync_copy(sw.at[pl.ds(0, K)], edata.at[pl.ds(off + 2 * K, K)])

    def flush_if_full(bkt, sr, sc, sw, cnt, nch):
        def do_flush(args):
            cnt, nch = args
            flush_chunk(bkt, sr, sc, sw, nch)
            tr = sr[pl.ds(K, 16)]
            tc_ = sc[pl.ds(K, 16)]
            tw = sw[pl.ds(K, 16)]
            sr[pl.ds(0, 16)] = tr
            sc[pl.ds(0, 16)] = tc_
            sw[pl.ds(0, 16)] = tw
            return cnt - K, nch + 1

        return lax.cond(cnt >= K, do_flush, lambda a: a, (cnt, nch))

    def block(j, carry):
        cnt0, nch0, cnt1, nch1 = carry
        off = ebase + j * BIN
        pltpu.sync_copy(row.at[pl.ds(off, BIN)], inr)
        pltpu.sync_copy(col.at[pl.ds(off, BIN)], inc)
        pltpu.sync_copy(w.at[pl.ds(off, BIN)], inw)

        def grp(g, gc):
            cnt0, nch0, cnt1, nch1 = gc
            p = g * 16
            valid = jnp.where(g < BIN // 16, 16, BIN - (BIN // 16) * 16)
            vm = iota < valid
            rv = inr[pl.ds(p, 16)]
            cv = inc[pl.ds(p, 16)]
            wvv = lax.bitcast_convert_type(inw[pl.ds(p, 16)], jnp.int32)

            m0 = vm & (cv < HALF)
            cs0 = plsc.cumsum(m0.astype(jnp.int32))
            i0 = jnp.where(m0, cnt0 + cs0 - 1, TRASH + iota)
            plsc.store_scatter(st0r, [i0], rv)
            plsc.store_scatter(st0c, [i0], cv)
            plsc.store_scatter(st0w, [i0], wvv)
            cnt0 = cnt0 + cs0[15]
            cnt0, nch0 = flush_if_full(0, st0r, st0c, st0w, cnt0, nch0)

            m1 = vm & (cv >= HALF)
            cs1 = plsc.cumsum(m1.astype(jnp.int32))
            i1 = jnp.where(m1, cnt1 + cs1 - 1, TRASH + iota)
            plsc.store_scatter(st1r, [i1], rv)
            plsc.store_scatter(st1c, [i1], cv - HALF)
            plsc.store_scatter(st1w, [i1], wvv)
            cnt1 = cnt1 + cs1[15]
            cnt1, nch1 = flush_if_full(1, st1r, st1c, st1w, cnt1, nch1)
            return (cnt0, nch0, cnt1, nch1)

        ngrp = BIN // 16 + (1 if BIN % 16 else 0)
        return lax.fori_loop(0, ngrp, grp, (cnt0, nch0, cnt1, nch1))

    z = jnp.int32(0)
    cnt0, nch0, cnt1, nch1 = lax.fori_loop(0, NBLK, block, (z, z, z, z))

    dz = jnp.zeros((16,), jnp.int32)
    dd = jnp.full((16,), DUMMY, jnp.int32)

    for bkt in range(2):
        cnt = (cnt0, cnt1)[bkt]
        nch = (nch0, nch1)[bkt]
        sr, sc, sw = ((st0r, st0c, st0w), (st1r, st1c, st1w))[bkt]
        for i in range(8):
            sr[pl.ds(cnt + i * 16, 16)] = dz
            sc[pl.ds(cnt + i * 16, 16)] = dd
            sw[pl.ds(cnt + i * 16, 16)] = dz

        @pl.when(cnt > 0)
        def _(bkt=bkt, sr=sr, sc=sc, sw=sw, nch=nch):
            flush_chunk(bkt, sr, sc, sw, nch)

        n = nch + jnp.where(cnt > 0, 1, 0)
        nv = jnp.full((16,), n, jnp.int32)
        jm = ((nv.astype(jnp.float32) + 4.5) * (1.0 / 3.0)).astype(jnp.int32)
        cbuf[pl.ds(0, 16)] = jnp.where(iota == 0, nv, jnp.where(iota == 1, jm, 0))
        pltpu.sync_copy(cbuf.at[pl.ds(0, 8)], ncnk.at[pl.ds((bkt * NW + t) * 8, 8)])


_partition = pl.kernel(
    _partition_body,
    out_type=(
        jax.ShapeDtypeStruct((2 * NW * SEGCH * CH_W,), jnp.int32),
        jax.ShapeDtypeStruct((2 * NW * 8,), jnp.int32),
    ),
    mesh=_mesh,
    compiler_params=pltpu.CompilerParams(use_tc_tiling_on_sc=False, needs_layout_passes=False),
    scratch_types=[
        pltpu.VMEM((BIN,), jnp.int32),
        pltpu.VMEM((BIN,), jnp.int32),
        pltpu.VMEM((BIN,), jnp.float32),
        pltpu.VMEM((STSZ,), jnp.int32),
        pltpu.VMEM((STSZ,), jnp.int32),
        pltpu.VMEM((STSZ,), jnp.int32),
        pltpu.VMEM((STSZ,), jnp.int32),
        pltpu.VMEM((STSZ,), jnp.int32),
        pltpu.VMEM((STSZ,), jnp.int32),
        pltpu.VMEM((16,), jnp.int32),
    ],
)


def _layer_body(x, edata, ncnk, zeros, out,
                acc, eb, cidx, rows, cbuf, esem, gsem, ssem):
    c = lax.axis_index("c")
    s = lax.axis_index("s")

    pltpu.sync_copy(zeros, acc.at[pl.ds(s * RPT, RPT)])

    pltpu.sync_copy(ncnk.at[pl.ds((c * NW + 2 * s) * 8, 8)], cbuf.at[pl.ds(0, 8)])
    pltpu.sync_copy(ncnk.at[pl.ds((c * NW + 2 * s + 1) * 8, 8)], cbuf.at[pl.ds(8, 8)])
    cv = cbuf[pl.ds(0, 16)]
    n0, jm0, n1, jm1 = cv[0], cv[1], cv[8], cv[9]

    plsc.subcore_barrier()

    def process_segment(t, n, jm):
        ebase = (c * NW + t) * SEGCH * CH_W

        def issue_edges(k, b):
            pltpu.async_copy(edata.at[pl.ds(ebase + k * CH_W, CH_W)],
                             eb.at[pl.ds(b * CH_W, CH_W)], esem.at[b])

        def wait_edges(b):
            pltpu.make_async_copy(edata.at[pl.ds(0, CH_W)],
                                  eb.at[pl.ds(b * CH_W, CH_W)], esem.at[b]).wait()

        def issue_gather(b):
            for g in range(K // 16):
                cidx[b, pl.ds(g * 16, 16)] = eb[pl.ds(b * CH_W + K + g * 16, 16)]
            pltpu.async_copy(x.at[eb.at[pl.ds(b * CH_W, K)]], rows.at[b], gsem.at[b])

        def wait_gather(b):
            pltpu.make_async_copy(x.at[eb.at[pl.ds(b * CH_W, K)]],
                                  rows.at[b], gsem.at[b]).wait()

        def scale_and_scatter(b):
            @plsc.parallel_loop(0, K, unroll=8)
            def _scale(e):
                wbits = jnp.full((16,), eb[pl.ds(b * CH_W + 2 * K + e, 16)][0], jnp.int32)
                wb = lax.bitcast_convert_type(wbits, jnp.float32)
                for d in range(DIM // 16):
                    rows[b, e, pl.ds(d * 16, 16)] = rows[b, e, pl.ds(d * 16, 16)] * wb

            pltpu.async_copy(rows.at[b], acc.at[cidx.at[b]], ssem.at[b], add=True)

        def wait_scatter(b):
            pltpu.make_async_copy(rows.at[b], acc.at[cidx.at[b]], ssem.at[b]).wait()

        def virt(j, carry):
            i0 = j * NBUF
            for b in range(NBUF):
                i = i0 + b
                ke = i
                kg = i - 1
                kp = i - 2

                @pl.when((ke >= NBUF) & (ke < n))
                def _():
                    wait_scatter(b)

                @pl.when(ke < n)
                def _():
                    issue_edges(ke, b)

                @pl.when((kg >= 0) & (kg < n))
                def _():
                    wait_edges((b + 2) % NBUF)
                    issue_gather((b + 2) % NBUF)

                @pl.when((kp >= 0) & (kp < n))
                def _():
                    wait_gather((b + 1) % NBUF)
                    scale_and_scatter((b + 1) % NBUF)

            return carry

        lax.fori_loop(0, jm, virt, 0)
        for b in range(NBUF):
            @pl.when(b < n)
            def _():
                wait_scatter(b)

    process_segment(2 * s, n0, jm0)
    process_segment(2 * s + 1, n1, jm1)

    plsc.subcore_barrier()

    base = s * RPT
    obase = c * HALF + s * RPT

    @pl.when(s < NS - 1)
    def _():
        pltpu.sync_copy(acc.at[pl.ds(base, RPT)], out.at[pl.ds(obase, RPT)])

    @pl.when(s == NS - 1)
    def _():
        pltpu.sync_copy(acc.at[pl.ds(base, RPT_LAST)], out.at[pl.ds(obase, RPT_LAST)])


_layer = pl.kernel(
    _layer_body,
    out_type=jax.ShapeDtypeStruct((N_NODES, DIM), jnp.float32),
    mesh=_mesh,
    compiler_params=pltpu.CompilerParams(use_tc_tiling_on_sc=False, needs_layout_passes=False),
    scratch_types=[
        pltpu.VMEM_SHARED((ACC_ROWS, DIM), jnp.float32),
        pltpu.VMEM((NBUF * CH_W + 16,), jnp.int32),
        pltpu.VMEM((NBUF, K), jnp.int32),
        pltpu.VMEM((NBUF, K, DIM), jnp.float32),
        pltpu.VMEM((16,), jnp.int32),
        pltpu.SemaphoreType.DMA((NBUF,)),
        pltpu.SemaphoreType.DMA((NBUF,)),
        pltpu.SemaphoreType.DMA((NBUF,)),
    ],
)

MTOT = N_NODES * DIM
MEPT = MTOT // (NC * NS)
MC = 10000
MNCH = MEPT // MC


def _mean_body(a0, a1, a2, a3, o, b0, b1, b2, b3, ob):
    c = lax.axis_index("c")
    s = lax.axis_index("s")
    base = (s * NC + c) * MEPT

    def chunk(j, carry):
        off = base + j * MC
        pltpu.sync_copy(a0.at[pl.ds(off, MC)], b0)
        pltpu.sync_copy(a1.at[pl.ds(off, MC)], b1)
        pltpu.sync_copy(a2.at[pl.ds(off, MC)], b2)
        pltpu.sync_copy(a3.at[pl.ds(off, MC)], b3)

        @plsc.parallel_loop(0, MC // 16, unroll=8)
        def grp(g):
            p = g * 16
            ob[pl.ds(p, 16)] = (
                b0[pl.ds(p, 16)] + b1[pl.ds(p, 16)] + b2[pl.ds(p, 16)] + b3[pl.ds(p, 16)]
            ) * 0.25

        pltpu.sync_copy(ob, o.at[pl.ds(off, MC)])
        return carry

    lax.fori_loop(0, MNCH, chunk, 0)


_mean = pl.kernel(
    _mean_body,
    out_type=jax.ShapeDtypeStruct((MTOT,), jnp.float32),
    mesh=_mesh,
    compiler_params=pltpu.CompilerParams(use_tc_tiling_on_sc=False, needs_layout_passes=False),
    scratch_types=[
        pltpu.VMEM((MC,), jnp.float32),
        pltpu.VMEM((MC,), jnp.float32),
        pltpu.VMEM((MC,), jnp.float32),
        pltpu.VMEM((MC,), jnp.float32),
        pltpu.VMEM((MC,), jnp.float32),
    ],
)


def kernel(embedding, edge_index, edge_weight):
    row = edge_index[0]
    col = edge_index[1]
    zeros = jnp.zeros((RPT, DIM), jnp.float32)
    edata, ncnk = _partition(row, col, edge_weight)
    x0 = embedding
    x1 = _layer(x0, edata, ncnk, zeros)
    x2 = _layer(x1, edata, ncnk, zeros)
    x3 = _layer(x2, edata, ncnk, zeros)
    of = _mean(x0.reshape(-1), x1.reshape(-1), x2.reshape(-1), x3.reshape(-1))
    return of.reshape(N_NODES, DIM)


# speedup vs baseline: 1.0893x; 1.0893x over previous
import jax
import jax.numpy as jnp
from jax import lax
from jax.experimental import pallas as pl
from jax.experimental.pallas import tpu as pltpu
from jax.experimental.pallas import tpu_sc as plsc

N_NODES = 50000
DIM = 64
N_EDGES = 800000

NC = 2
NS = 16
NW = NC * NS
HALF = N_NODES // NC
DUMMY = HALF
RPT = 1568
ACC_ROWS = RPT * NS
RPT_LAST = HALF - (NS - 1) * RPT
K = 80
CH_W = 3 * K
NBUF = 5
PEPT = N_EDGES // NW
SEGCH = (PEPT + K - 1) // K
BIN = 1000
NBLK = PEPT // BIN
STSZ = 224
TRASH = 208

_mesh = plsc.VectorSubcoreMesh(core_axis_name="c", subcore_axis_name="s")


def _partition_body(row, col, w, edata, ncnk,
                    inr, inc, inw, st0r, st0c, st0w, st1r, st1c, st1w, cbuf):
    c = lax.axis_index("c")
    s = lax.axis_index("s")
    t = c * NS + s
    ebase = t * PEPT

    iota = lax.iota(jnp.int32, 16)

    def seg_off(bkt):
        return (bkt * NW + t) * SEGCH * CH_W

    def flush_chunk(bkt, sr, sc, sw, nch):
        off = seg_off(bkt) + nch * CH_W
        pltpu.sync_copy(sr.at[pl.ds(0, K)], edata.at[pl.ds(off, K)])
        pltpu.sync_copy(sc.at[pl.ds(0, K)], edata.at[pl.ds(off + K, K)])
        pltpu.sync_copy(sw.at[pl.ds(0, K)], edata.at[pl.ds(off + 2 * K, K)])

    def flush_if_full(bkt, sr, sc, sw, cnt, nch):
        def do_flush(args):
            cnt, nch = args
            flush_chunk(bkt, sr, sc, sw, nch)
            tr = sr[pl.ds(K, 16)]
            tc_ = sc[pl.ds(K, 16)]
            tw = sw[pl.ds(K, 16)]
            sr[pl.ds(0, 16)] = tr
            sc[pl.ds(0, 16)] = tc_
            sw[pl.ds(0, 16)] = tw
            return cnt - K, nch + 1

        return lax.cond(cnt >= K, do_flush, lambda a: a, (cnt, nch))

    def block(j, carry):
        cnt0, nch0, cnt1, nch1 = carry
        off = ebase + j * BIN
        pltpu.sync_copy(row.at[pl.ds(off, BIN)], inr)
        pltpu.sync_copy(col.at[pl.ds(off, BIN)], inc)
        pltpu.sync_copy(w.at[pl.ds(off, BIN)], inw)

        def grp(g, gc):
            cnt0, nch0, cnt1, nch1 = gc
            p = g * 16
            valid = jnp.where(g < BIN // 16, 16, BIN - (BIN // 16) * 16)
            vm = iota < valid
            rv = inr[pl.ds(p, 16)]
            cv = inc[pl.ds(p, 16)]
            wvv = lax.bitcast_convert_type(inw[pl.ds(p, 16)], jnp.int32)

            m0 = vm & (cv < HALF)
            cs0 = plsc.cumsum(m0.astype(jnp.int32))
            i0 = jnp.where(m0, cnt0 + cs0 - 1, TRASH + iota)
            plsc.store_scatter(st0r, [i0], rv)
            plsc.store_scatter(st0c, [i0], cv)
            plsc.store_scatter(st0w, [i0], wvv)
            cnt0 = cnt0 + cs0[15]
            cnt0, nch0 = flush_if_full(0, st0r, st0c, st0w, cnt0, nch0)

            m1 = vm & (cv >= HALF)
            cs1 = plsc.cumsum(m1.astype(jnp.int32))
            i1 = jnp.where(m1, cnt1 + cs1 - 1, TRASH + iota)
            plsc.store_scatter(st1r, [i1], rv)
            plsc.store_scatter(st1c, [i1], cv - HALF)
            plsc.store_scatter(st1w, [i1], wvv)
            cnt1 = cnt1 + cs1[15]
            cnt1, nch1 = flush_if_full(1, st1r, st1c, st1w, cnt1, nch1)
            return (cnt0, nch0, cnt1, nch1)

        ngrp = BIN // 16 + (1 if BIN % 16 else 0)
        return lax.fori_loop(0, ngrp, grp, (cnt0, nch0, cnt1, nch1))

    z = jnp.int32(0)
    cnt0, nch0, cnt1, nch1 = lax.fori_loop(0, NBLK, block, (z, z, z, z))

    dz = jnp.zeros((16,), jnp.int32)
    dd = jnp.full((16,), DUMMY, jnp.int32)

    for bkt in range(2):
        cnt = (cnt0, cnt1)[bkt]
        nch = (nch0, nch1)[bkt]
        sr, sc, sw = ((st0r, st0c, st0w), (st1r, st1c, st1w))[bkt]
        for i in range(8):
            sr[pl.ds(cnt + i * 16, 16)] = dz
            sc[pl.ds(cnt + i * 16, 16)] = dd
            sw[pl.ds(cnt + i * 16, 16)] = dz

        @pl.when(cnt > 0)
        def _(bkt=bkt, sr=sr, sc=sc, sw=sw, nch=nch):
            flush_chunk(bkt, sr, sc, sw, nch)

        n = nch + jnp.where(cnt > 0, 1, 0)
        nv = jnp.full((16,), n, jnp.int32)
        jm = ((nv.astype(jnp.float32) + 8.5) * 0.2).astype(jnp.int32)
        cbuf[pl.ds(0, 16)] = jnp.where(iota == 0, nv, jnp.where(iota == 1, jm, 0))
        pltpu.sync_copy(cbuf.at[pl.ds(0, 8)], ncnk.at[pl.ds((bkt * NW + t) * 8, 8)])


_partition = pl.kernel(
    _partition_body,
    out_type=(
        jax.ShapeDtypeStruct((2 * NW * SEGCH * CH_W,), jnp.int32),
        jax.ShapeDtypeStruct((2 * NW * 8,), jnp.int32),
    ),
    mesh=_mesh,
    compiler_params=pltpu.CompilerParams(use_tc_tiling_on_sc=False, needs_layout_passes=False),
    scratch_types=[
        pltpu.VMEM((BIN,), jnp.int32),
        pltpu.VMEM((BIN,), jnp.int32),
        pltpu.VMEM((BIN,), jnp.float32),
        pltpu.VMEM((STSZ,), jnp.int32),
        pltpu.VMEM((STSZ,), jnp.int32),
        pltpu.VMEM((STSZ,), jnp.int32),
        pltpu.VMEM((STSZ,), jnp.int32),
        pltpu.VMEM((STSZ,), jnp.int32),
        pltpu.VMEM((STSZ,), jnp.int32),
        pltpu.VMEM((16,), jnp.int32),
    ],
)


def _layer_body(x, edata, ncnk, zeros, out,
                acc, eb, cidx, rows, cbuf, esem, gsem, ssem):
    c = lax.axis_index("c")
    s = lax.axis_index("s")

    pltpu.sync_copy(zeros, acc.at[pl.ds(s * RPT, RPT)])

    pltpu.sync_copy(ncnk.at[pl.ds((c * NW + 2 * s) * 8, 8)], cbuf.at[pl.ds(0, 8)])
    pltpu.sync_copy(ncnk.at[pl.ds((c * NW + 2 * s + 1) * 8, 8)], cbuf.at[pl.ds(8, 8)])
    cv = cbuf[pl.ds(0, 16)]
    n0, jm0, n1, jm1 = cv[0], cv[1], cv[8], cv[9]

    plsc.subcore_barrier()

    def process_segment(t, n, jm):
        ebase = (c * NW + t) * SEGCH * CH_W

        def issue_edges(k, b):
            pltpu.async_copy(edata.at[pl.ds(ebase + k * CH_W, CH_W)],
                             eb.at[pl.ds(b * CH_W, CH_W)], esem.at[b])

        def wait_edges(b):
            pltpu.make_async_copy(edata.at[pl.ds(0, CH_W)],
                                  eb.at[pl.ds(b * CH_W, CH_W)], esem.at[b]).wait()

        def issue_gather(b):
            for g in range(K // 16):
                cidx[b, pl.ds(g * 16, 16)] = eb[pl.ds(b * CH_W + K + g * 16, 16)]
            pltpu.async_copy(x.at[eb.at[pl.ds(b * CH_W, K)]], rows.at[b], gsem.at[b])

        def wait_gather(b):
            pltpu.make_async_copy(x.at[eb.at[pl.ds(b * CH_W, K)]],
                                  rows.at[b], gsem.at[b]).wait()

        def scale_and_scatter(b):
            @plsc.parallel_loop(0, K, unroll=8)
            def _scale(e):
                wbits = jnp.full((16,), eb[pl.ds(b * CH_W + 2 * K + e, 16)][0], jnp.int32)
                wb = lax.bitcast_convert_type(wbits, jnp.float32)
                for d in range(DIM // 16):
                    rows[b, e, pl.ds(d * 16, 16)] = rows[b, e, pl.ds(d * 16, 16)] * wb

            pltpu.async_copy(rows.at[b], acc.at[cidx.at[b]], ssem.at[b], add=True)

        def wait_scatter(b):
            pltpu.make_async_copy(rows.at[b], acc.at[cidx.at[b]], ssem.at[b]).wait()

        def virt(j, carry):
            i0 = j * NBUF
            for b in range(NBUF):
                i = i0 + b
                ke = i
                kg = i - 2
                kp = i - 4

                @pl.when((ke >= NBUF) & (ke < n))
                def _():
                    wait_scatter(b)

                @pl.when(ke < n)
                def _():
                    issue_edges(ke, b)

                @pl.when((kg >= 0) & (kg < n))
                def _():
                    wait_edges((b + 3) % NBUF)
                    issue_gather((b + 3) % NBUF)

                @pl.when((kp >= 0) & (kp < n))
                def _():
                    wait_gather((b + 1) % NBUF)
                    scale_and_scatter((b + 1) % NBUF)

            return carry

        lax.fori_loop(0, jm, virt, 0)
        for b in range(NBUF):
            @pl.when(b < n)
            def _():
                wait_scatter(b)

    process_segment(2 * s, n0, jm0)
    process_segment(2 * s + 1, n1, jm1)

    plsc.subcore_barrier()

    base = s * RPT
    obase = c * HALF + s * RPT

    @pl.when(s < NS - 1)
    def _():
        pltpu.sync_copy(acc.at[pl.ds(base, RPT)], out.at[pl.ds(obase, RPT)])

    @pl.when(s == NS - 1)
    def _():
        pltpu.sync_copy(acc.at[pl.ds(base, RPT_LAST)], out.at[pl.ds(obase, RPT_LAST)])


_layer = pl.kernel(
    _layer_body,
    out_type=jax.ShapeDtypeStruct((N_NODES, DIM), jnp.float32),
    mesh=_mesh,
    compiler_params=pltpu.CompilerParams(use_tc_tiling_on_sc=False, needs_layout_passes=False),
    scratch_types=[
        pltpu.VMEM_SHARED((ACC_ROWS, DIM), jnp.float32),
        pltpu.VMEM((NBUF * CH_W + 16,), jnp.int32),
        pltpu.VMEM((NBUF, K), jnp.int32),
        pltpu.VMEM((NBUF, K, DIM), jnp.float32),
        pltpu.VMEM((16,), jnp.int32),
        pltpu.SemaphoreType.DMA((NBUF,)),
        pltpu.SemaphoreType.DMA((NBUF,)),
        pltpu.SemaphoreType.DMA((NBUF,)),
    ],
)

MTOT = N_NODES * DIM
MEPT = MTOT // (NC * NS)
MC = 10000
MNCH = MEPT // MC


def _mean_body(a0, a1, a2, a3, o, b0, b1, b2, b3, ob, msem):
    c = lax.axis_index("c")
    s = lax.axis_index("s")
    base = (s * NC + c) * MEPT

    def chunk(j, carry):
        off = base + j * MC
        cp0 = pltpu.async_copy(a0.at[pl.ds(off, MC)], b0, msem)
        cp1 = pltpu.async_copy(a1.at[pl.ds(off, MC)], b1, msem)
        cp2 = pltpu.async_copy(a2.at[pl.ds(off, MC)], b2, msem)
        cp3 = pltpu.async_copy(a3.at[pl.ds(off, MC)], b3, msem)
        cp0.wait()
        cp1.wait()
        cp2.wait()
        cp3.wait()

        @plsc.parallel_loop(0, MC // 16, unroll=8)
        def grp(g):
            p = g * 16
            ob[pl.ds(p, 16)] = (
                b0[pl.ds(p, 16)] + b1[pl.ds(p, 16)] + b2[pl.ds(p, 16)] + b3[pl.ds(p, 16)]
            ) * 0.25

        pltpu.sync_copy(ob, o.at[pl.ds(off, MC)])
        return carry

    lax.fori_loop(0, MNCH, chunk, 0)


_mean = pl.kernel(
    _mean_body,
    out_type=jax.ShapeDtypeStruct((MTOT,), jnp.float32),
    mesh=_mesh,
    compiler_params=pltpu.CompilerParams(use_tc_tiling_on_sc=False, needs_layout_passes=False),
    scratch_types=[
        pltpu.VMEM((MC,), jnp.float32),
        pltpu.VMEM((MC,), jnp.float32),
        pltpu.VMEM((MC,), jnp.float32),
        pltpu.VMEM((MC,), jnp.float32),
        pltpu.VMEM((MC,), jnp.float32),
        pltpu.SemaphoreType.DMA,
    ],
)


def kernel(embedding, edge_index, edge_weight):
    row = edge_index[0]
    col = edge_index[1]
    zeros = jnp.zeros((RPT, DIM), jnp.float32)
    edata, ncnk = _partition(row, col, edge_weight)
    x0 = embedding
    x1 = _layer(x0, edata, ncnk, zeros)
    x2 = _layer(x1, edata, ncnk, zeros)
    x3 = _layer(x2, edata, ncnk, zeros)
    of = _mean(x0.reshape(-1), x1.reshape(-1), x2.reshape(-1), x3.reshape(-1))
    return of.reshape(N_NODES, DIM)


# speedup vs baseline: 1.2516x; 1.1490x over previous
import jax
import jax.numpy as jnp
from jax import lax
from jax.experimental import pallas as pl
from jax.experimental.pallas import tpu as pltpu
from jax.experimental.pallas import tpu_sc as plsc

N_NODES = 50000
DIM = 64
N_EDGES = 800000

NC = 2
NS = 16
NW = NC * NS
HALF = N_NODES // NC
DUMMY = HALF
RPT = 1568
ACC_ROWS = RPT * NS
RPT_LAST = HALF - (NS - 1) * RPT
K = 80
CH_W = 3 * K
NBUF = 5
PEPT = N_EDGES // NW
SEGCH = (PEPT + K - 1) // K
BIN = 5000
NBLK = PEPT // BIN
NGRP = (BIN + 15) // 16
MAXFL = 64
TRASH = 5088
STSZ = 5104

_mesh = plsc.VectorSubcoreMesh(core_axis_name="c", subcore_axis_name="s")


def _partition_body(row, col, w, edata, ncnk,
                    inr, inc, inw, st0r, st0c, st0w, st1r, st1c, st1w, cbuf, fsem):
    c = lax.axis_index("c")
    s = lax.axis_index("s")
    t = c * NS + s
    ebase = t * PEPT

    iota = lax.iota(jnp.int32, 16)

    def seg_off(bkt):
        return (bkt * NW + t) * SEGCH * CH_W

    def flush_chunk_sync(bkt, sr, sc, sw, src_ch, nch):
        off = seg_off(bkt) + nch * CH_W
        pltpu.sync_copy(sr.at[pl.ds(src_ch * K, K)], edata.at[pl.ds(off, K)])
        pltpu.sync_copy(sc.at[pl.ds(src_ch * K, K)], edata.at[pl.ds(off + K, K)])
        pltpu.sync_copy(sw.at[pl.ds(src_ch * K, K)], edata.at[pl.ds(off + 2 * K, K)])

    def block(j, carry):
        cnt0, nch0, cnt1, nch1 = carry
        off = ebase + j * BIN
        cp0 = pltpu.async_copy(row.at[pl.ds(off, BIN)], inr, fsem)
        cp1 = pltpu.async_copy(col.at[pl.ds(off, BIN)], inc, fsem)
        cp2 = pltpu.async_copy(w.at[pl.ds(off, BIN)], inw, fsem)
        cp0.wait()
        cp1.wait()
        cp2.wait()

        def grp(g, gc):
            cnt0, cnt1 = gc
            p = g * 16
            valid = jnp.where(g < BIN // 16, 16, BIN - (BIN // 16) * 16)
            vm = iota < valid
            rv = inr[pl.ds(p, 16)]
            cv = inc[pl.ds(p, 16)]
            wvv = lax.bitcast_convert_type(inw[pl.ds(p, 16)], jnp.int32)

            m0 = vm & (cv < HALF)
            cs0 = plsc.cumsum(m0.astype(jnp.int32))
            i0 = jnp.where(m0, cnt0 + cs0 - 1, TRASH + iota)
            plsc.store_scatter(st0r, [i0], rv)
            plsc.store_scatter(st0c, [i0], cv)
            plsc.store_scatter(st0w, [i0], wvv)

            m1 = vm & (cv >= HALF)
            cs1 = plsc.cumsum(m1.astype(jnp.int32))
            i1 = jnp.where(m1, cnt1 + cs1 - 1, TRASH + iota)
            plsc.store_scatter(st1r, [i1], rv)
            plsc.store_scatter(st1c, [i1], cv - HALF)
            plsc.store_scatter(st1w, [i1], wvv)
            return (cnt0 + cs0[15], cnt1 + cs1[15])

        cnt0, cnt1 = lax.fori_loop(0, NGRP, grp, (cnt0, cnt1), unroll=2)

        for bkt in range(2):
            cnt = (cnt0, cnt1)[bkt]
            nch = (nch0, nch1)[bkt]
            sr, sc, sw = ((st0r, st0c, st0w), (st1r, st1c, st1w))[bkt]

            def fire(i, a, bkt=bkt, sr=sr, sc=sc, sw=sw, cnt=cnt, nch=nch):
                @pl.when((i + 1) * K <= cnt)
                def _():
                    off = seg_off(bkt) + (nch + i) * CH_W
                    pltpu.async_copy(sr.at[pl.ds(i * K, K)], edata.at[pl.ds(off, K)], fsem)
                    pltpu.async_copy(sc.at[pl.ds(i * K, K)], edata.at[pl.ds(off + K, K)], fsem)
                    pltpu.async_copy(sw.at[pl.ds(i * K, K)], edata.at[pl.ds(off + 2 * K, K)], fsem)

                return a + jnp.where((i + 1) * K <= cnt, 1, 0)

            nfl = lax.fori_loop(0, MAXFL, fire, jnp.int32(0))

            def drain(i, a, sr=sr, sc=sc, sw=sw, nfl=nfl):
                @pl.when(i < nfl)
                def _():
                    pltpu.make_async_copy(sr.at[pl.ds(0, K)], edata.at[pl.ds(0, K)], fsem).wait()
                    pltpu.make_async_copy(sc.at[pl.ds(0, K)], edata.at[pl.ds(0, K)], fsem).wait()
                    pltpu.make_async_copy(sw.at[pl.ds(0, K)], edata.at[pl.ds(0, K)], fsem).wait()

                return a

            lax.fori_loop(0, MAXFL, drain, jnp.int32(0))

            pbase = nfl * K
            for i in range(K // 16):
                sr[pl.ds(i * 16, 16)] = sr[pl.ds(pbase + i * 16, 16)]
                sc[pl.ds(i * 16, 16)] = sc[pl.ds(pbase + i * 16, 16)]
                sw[pl.ds(i * 16, 16)] = sw[pl.ds(pbase + i * 16, 16)]

            if bkt == 0:
                cnt0 = cnt - pbase
                nch0 = nch + nfl
            else:
                cnt1 = cnt - pbase
                nch1 = nch + nfl

        return (cnt0, nch0, cnt1, nch1)

    z = jnp.int32(0)
    cnt0, nch0, cnt1, nch1 = lax.fori_loop(0, NBLK, block, (z, z, z, z))

    dz = jnp.zeros((16,), jnp.int32)
    dd = jnp.full((16,), DUMMY, jnp.int32)

    for bkt in range(2):
        cnt = (cnt0, cnt1)[bkt]
        nch = (nch0, nch1)[bkt]
        sr, sc, sw = ((st0r, st0c, st0w), (st1r, st1c, st1w))[bkt]
        for i in range(8):
            sr[pl.ds(cnt + i * 16, 16)] = dz
            sc[pl.ds(cnt + i * 16, 16)] = dd
            sw[pl.ds(cnt + i * 16, 16)] = dz

        @pl.when(cnt > 0)
        def _(bkt=bkt, sr=sr, sc=sc, sw=sw, nch=nch):
            flush_chunk_sync(bkt, sr, sc, sw, 0, nch)

        n = nch + jnp.where(cnt > 0, 1, 0)
        nv = jnp.full((16,), n, jnp.int32)
        jm = ((nv.astype(jnp.float32) + 8.5) * 0.2).astype(jnp.int32)
        cbuf[pl.ds(0, 16)] = jnp.where(iota == 0, nv, jnp.where(iota == 1, jm, 0))
        pltpu.sync_copy(cbuf.at[pl.ds(0, 8)], ncnk.at[pl.ds((bkt * NW + t) * 8, 8)])


_partition = pl.kernel(
    _partition_body,
    out_type=(
        jax.ShapeDtypeStruct((2 * NW * SEGCH * CH_W,), jnp.int32),
        jax.ShapeDtypeStruct((2 * NW * 8,), jnp.int32),
    ),
    mesh=_mesh,
    compiler_params=pltpu.CompilerParams(use_tc_tiling_on_sc=False, needs_layout_passes=False),
    scratch_types=[
        pltpu.VMEM((BIN,), jnp.int32),
        pltpu.VMEM((BIN,), jnp.int32),
        pltpu.VMEM((BIN,), jnp.float32),
        pltpu.VMEM((STSZ,), jnp.int32),
        pltpu.VMEM((STSZ,), jnp.int32),
        pltpu.VMEM((STSZ,), jnp.int32),
        pltpu.VMEM((STSZ,), jnp.int32),
        pltpu.VMEM((STSZ,), jnp.int32),
        pltpu.VMEM((STSZ,), jnp.int32),
        pltpu.VMEM((16,), jnp.int32),
        pltpu.SemaphoreType.DMA,
    ],
)


def _layer_body(x, edata, ncnk, zeros, out,
                acc, eb, cidx, rows, cbuf, esem, gsem, ssem):
    c = lax.axis_index("c")
    s = lax.axis_index("s")

    pltpu.sync_copy(zeros, acc.at[pl.ds(s * RPT, RPT)])

    pltpu.sync_copy(ncnk.at[pl.ds((c * NW + 2 * s) * 8, 8)], cbuf.at[pl.ds(0, 8)])
    pltpu.sync_copy(ncnk.at[pl.ds((c * NW + 2 * s + 1) * 8, 8)], cbuf.at[pl.ds(8, 8)])
    cv = cbuf[pl.ds(0, 16)]
    n0, jm0, n1, jm1 = cv[0], cv[1], cv[8], cv[9]

    plsc.subcore_barrier()

    def process_segment(t, n, jm):
        ebase = (c * NW + t) * SEGCH * CH_W

        def issue_edges(k, b):
            pltpu.async_copy(edata.at[pl.ds(ebase + k * CH_W, CH_W)],
                             eb.at[pl.ds(b * CH_W, CH_W)], esem.at[b])

        def wait_edges(b):
            pltpu.make_async_copy(edata.at[pl.ds(0, CH_W)],
                                  eb.at[pl.ds(b * CH_W, CH_W)], esem.at[b]).wait()

        def issue_gather(b):
            for g in range(K // 16):
                cidx[b, pl.ds(g * 16, 16)] = eb[pl.ds(b * CH_W + K + g * 16, 16)]
            pltpu.async_copy(x.at[eb.at[pl.ds(b * CH_W, K)]], rows.at[b], gsem.at[b])

        def wait_gather(b):
            pltpu.make_async_copy(x.at[eb.at[pl.ds(b * CH_W, K)]],
                                  rows.at[b], gsem.at[b]).wait()

        def scale_and_scatter(b):
            @plsc.parallel_loop(0, K, unroll=8)
            def _scale(e):
                wbits = jnp.full((16,), eb[pl.ds(b * CH_W + 2 * K + e, 16)][0], jnp.int32)
                wb = lax.bitcast_convert_type(wbits, jnp.float32)
                for d in range(DIM // 16):
                    rows[b, e, pl.ds(d * 16, 16)] = rows[b, e, pl.ds(d * 16, 16)] * wb

            pltpu.async_copy(rows.at[b], acc.at[cidx.at[b]], ssem.at[b], add=True)

        def wait_scatter(b):
            pltpu.make_async_copy(rows.at[b], acc.at[cidx.at[b]], ssem.at[b]).wait()

        def virt(j, carry):
            i0 = j * NBUF
            for b in range(NBUF):
                i = i0 + b
                ke = i
                kg = i - 2
                kp = i - 4

                @pl.when((ke >= NBUF) & (ke < n))
                def _():
                    wait_scatter(b)

                @pl.when(ke < n)
                def _():
                    issue_edges(ke, b)

                @pl.when((kg >= 0) & (kg < n))
                def _():
                    wait_edges((b + 3) % NBUF)
                    issue_gather((b + 3) % NBUF)

                @pl.when((kp >= 0) & (kp < n))
                def _():
                    wait_gather((b + 1) % NBUF)
                    scale_and_scatter((b + 1) % NBUF)

            return carry

        lax.fori_loop(0, jm, virt, 0)
        for b in range(NBUF):
            @pl.when(b < n)
            def _():
                wait_scatter(b)

    process_segment(2 * s, n0, jm0)
    process_segment(2 * s + 1, n1, jm1)

    plsc.subcore_barrier()

    base = s * RPT
    obase = c * HALF + s * RPT

    @pl.when(s < NS - 1)
    def _():
        pltpu.sync_copy(acc.at[pl.ds(base, RPT)], out.at[pl.ds(obase, RPT)])

    @pl.when(s == NS - 1)
    def _():
        pltpu.sync_copy(acc.at[pl.ds(base, RPT_LAST)], out.at[pl.ds(obase, RPT_LAST)])


_layer = pl.kernel(
    _layer_body,
    out_type=jax.ShapeDtypeStruct((N_NODES, DIM), jnp.float32),
    mesh=_mesh,
    compiler_params=pltpu.CompilerParams(use_tc_tiling_on_sc=False, needs_layout_passes=False),
    scratch_types=[
        pltpu.VMEM_SHARED((ACC_ROWS, DIM), jnp.float32),
        pltpu.VMEM((NBUF * CH_W + 16,), jnp.int32),
        pltpu.VMEM((NBUF, K), jnp.int32),
        pltpu.VMEM((NBUF, K, DIM), jnp.float32),
        pltpu.VMEM((16,), jnp.int32),
        pltpu.SemaphoreType.DMA((NBUF,)),
        pltpu.SemaphoreType.DMA((NBUF,)),
        pltpu.SemaphoreType.DMA((NBUF,)),
    ],
)

MTOT = N_NODES * DIM
MEPT = MTOT // (NC * NS)
MC = 10000
MNCH = MEPT // MC


def _mean_body(a0, a1, a2, a3, o, b0, b1, b2, b3, ob, msem):
    c = lax.axis_index("c")
    s = lax.axis_index("s")
    base = (s * NC + c) * MEPT

    def chunk(j, carry):
        off = base + j * MC
        cp0 = pltpu.async_copy(a0.at[pl.ds(off, MC)], b0, msem)
        cp1 = pltpu.async_copy(a1.at[pl.ds(off, MC)], b1, msem)
        cp2 = pltpu.async_copy(a2.at[pl.ds(off, MC)], b2, msem)
        cp3 = pltpu.async_copy(a3.at[pl.ds(off, MC)], b3, msem)
        cp0.wait()
        cp1.wait()
        cp2.wait()
        cp3.wait()

        @plsc.parallel_loop(0, MC // 16, unroll=8)
        def grp(g):
            p = g * 16
            ob[pl.ds(p, 16)] = (
                b0[pl.ds(p, 16)] + b1[pl.ds(p, 16)] + b2[pl.ds(p, 16)] + b3[pl.ds(p, 16)]
            ) * 0.25

        pltpu.sync_copy(ob, o.at[pl.ds(off, MC)])
        return carry

    lax.fori_loop(0, MNCH, chunk, 0)


_mean = pl.kernel(
    _mean_body,
    out_type=jax.ShapeDtypeStruct((MTOT,), jnp.float32),
    mesh=_mesh,
    compiler_params=pltpu.CompilerParams(use_tc_tiling_on_sc=False, needs_layout_passes=False),
    scratch_types=[
        pltpu.VMEM((MC,), jnp.float32),
        pltpu.VMEM((MC,), jnp.float32),
        pltpu.VMEM((MC,), jnp.float32),
        pltpu.VMEM((MC,), jnp.float32),
        pltpu.VMEM((MC,), jnp.float32),
        pltpu.SemaphoreType.DMA,
    ],
)


def kernel(embedding, edge_index, edge_weight):
    row = edge_index[0]
    col = edge_index[1]
    zeros = jnp.zeros((RPT, DIM), jnp.float32)
    edata, ncnk = _partition(row, col, edge_weight)
    x0 = embedding
    x1 = _layer(x0, edata, ncnk, zeros)
    x2 = _layer(x1, edata, ncnk, zeros)
    x3 = _layer(x2, edata, ncnk, zeros)
    of = _mean(x0.reshape(-1), x1.reshape(-1), x2.reshape(-1), x3.reshape(-1))
    return of.reshape(N_NODES, DIM)
